# baseline (speedup 1.0000x reference)
import jax
import jax.numpy as jnp
from jax import lax
from jax.experimental import pallas as pl
from jax.experimental.pallas import tpu as pltpu
from jax.experimental.pallas import tpu_sc as plsc

_NUM_CORES = 2
_NUM_SUBCORES = 16
_NUM_WORKERS = _NUM_CORES * _NUM_SUBCORES
_LANES = 16


def _make_body(rows_per_worker, words_per_row, n_out):
    groups = rows_per_worker // _LANES

    def body(ids_hbm, coef_hbm, out_hbm, buf, coefv, outv):
        cid = lax.axis_index("c")
        sid = lax.axis_index("s")
        wid = sid * _NUM_CORES + cid
        base = wid * rows_per_worker
        pltpu.sync_copy(
            ids_hbm.at[pl.ds(base * words_per_row, rows_per_worker * words_per_row)],
            buf,
        )
        pltpu.sync_copy(coef_hbm, coefv)
        lane = lax.iota(jnp.int32, 16)

        byte_mask = jnp.full((16,), 0x01010101, jnp.int32)
        byte_sum = jnp.full((16,), 0x01010101, jnp.int32)

        def group(g, _):
            rows = g * _LANES + lane
            elt0 = rows * words_per_row
            zero = jnp.zeros((16,), jnp.int32)
            s0 = s1 = s01 = zero
            for st in range(words_per_row):
                c = plsc.load_gather(buf, [elt0 + st])
                t0 = c & byte_mask
                t1 = (c >> 1) & byte_mask
                s0 = s0 + t0
                s1 = s1 + t1
                s01 = s01 + (t0 & t1)

            def byte_total(v):
                return lax.shift_right_logical(v * byte_sum, 24).astype(jnp.float32)

            f0 = byte_total(s0)
            f1 = byte_total(s1)
            f01 = byte_total(s01)
            out0 = rows * n_out
            for j in range(3):
                v = coefv[pl.ds((4 * j) * 16, 16)] + coefv[pl.ds((4 * j + 1) * 16, 16)] * f0
                v = v + coefv[pl.ds((4 * j + 2) * 16, 16)] * f1 + coefv[pl.ds((4 * j + 3) * 16, 16)] * f01
                plsc.store_scatter(outv, [out0 + j], v)
            return 0

        lax.fori_loop(0, groups, group, 0)
        pltpu.sync_copy(outv, out_hbm.at[pl.ds(base * n_out, rows_per_worker * n_out)])

    return body


def kernel(input_ids, embedding_table, dense_w):
    batch, seq_len = input_ids.shape
    n_out = dense_w.shape[1]
    assert batch % (_NUM_WORKERS * _LANES) == 0
    assert seq_len % 4 == 0
    words_per_row = seq_len // 4
    rows_per_worker = batch // _NUM_WORKERS

    m = embedding_table.astype(jnp.float32) @ dense_w.astype(jnp.float32)
    a = m[0]
    b = m[1] - m[0]
    c = m[2] - m[0]
    d = m[3] - m[1] - m[2] + m[0]
    k = seq_len * a
    coef = jnp.stack([k, b, c, d], axis=0).T.reshape(4 * n_out)
    coef = jnp.broadcast_to(coef[:, None], (4 * n_out, _LANES)).reshape(-1)

    ids = input_ids.astype(jnp.int32)
    packed = (
        ids[:, 0::4]
        | (ids[:, 1::4] << 8)
        | (ids[:, 2::4] << 16)
        | (ids[:, 3::4] << 24)
    )
    packed = packed.reshape(batch * words_per_row)

    fn = pl.kernel(
        _make_body(rows_per_worker, words_per_row, n_out),
        out_type=jax.ShapeDtypeStruct((batch * n_out,), jnp.float32),
        mesh=plsc.VectorSubcoreMesh(
            core_axis_name="c",
            subcore_axis_name="s",
            num_cores=_NUM_CORES,
            num_subcores=_NUM_SUBCORES,
        ),
        scratch_types=[
            pltpu.VMEM((rows_per_worker * words_per_row,), jnp.int32),
            pltpu.VMEM((4 * n_out * _LANES,), jnp.float32),
            pltpu.VMEM((rows_per_worker * n_out,), jnp.float32),
        ],
        compiler_params=pltpu.CompilerParams(
            use_tc_tiling_on_sc=False, needs_layout_passes=False
        ),
    )
    return fn(packed, coef).reshape(batch, n_out)


# speedup vs baseline: 1.6612x; 1.6612x over previous
import jax
import jax.numpy as jnp
from jax import lax
from jax.experimental import pallas as pl
from jax.experimental.pallas import tpu as pltpu
from jax.experimental.pallas import tpu_sc as plsc

_NUM_CORES = 2
_NUM_SUBCORES = 16
_NUM_WORKERS = _NUM_CORES * _NUM_SUBCORES
_LANES = 16
_CHUNK_ROWS = 128


def _make_body(rows_per_worker, seq_len, n_out):
    n_chunks = rows_per_worker // _CHUNK_ROWS
    groups_per_chunk = _CHUNK_ROWS // _LANES

    def body(ids_hbm, coef_hbm, out_hbm, buf0, buf1, coefv, outv, sem0, sem1):
        cid = lax.axis_index("c")
        sid = lax.axis_index("s")
        wid = sid * _NUM_CORES + cid
        base = wid * rows_per_worker
        pltpu.sync_copy(coef_hbm, coefv)
        lane = lax.iota(jnp.int32, 16)

        byte_mask = jnp.full((16,), 0x01010101, jnp.int32)
        byte_sum = jnp.full((16,), 0x01010101, jnp.int32)
        bufs = (buf0, buf1)
        sems = (sem0, sem1)

        def copy_chunk(ch):
            return pltpu.make_async_copy(
                ids_hbm.at[pl.ds(base + ch * _CHUNK_ROWS, _CHUNK_ROWS)],
                bufs[ch % 2],
                sems[ch % 2],
            )

        copy_chunk(0).start()
        for ch in range(n_chunks):
            if ch + 1 < n_chunks:
                copy_chunk(ch + 1).start()
            copy_chunk(ch).wait()
            buf = bufs[ch % 2]

            def group(g, _, buf=buf, ch=ch):
                rows = g * _LANES + lane
                zero = jnp.zeros((16,), jnp.int32)
                s0 = s1 = s01 = zero
                for st in range(seq_len // 4):
                    l = 4 * st
                    x0 = plsc.load_gather(buf, [rows, jnp.full((16,), l, jnp.int32)])
                    x1 = plsc.load_gather(buf, [rows, jnp.full((16,), l + 1, jnp.int32)])
                    x2 = plsc.load_gather(buf, [rows, jnp.full((16,), l + 2, jnp.int32)])
                    x3 = plsc.load_gather(buf, [rows, jnp.full((16,), l + 3, jnp.int32)])
                    c = x0 | (x1 << 8) | (x2 << 16) | (x3 << 24)
                    t0 = c & byte_mask
                    t1 = (c >> 1) & byte_mask
                    s0 = s0 + t0
                    s1 = s1 + t1
                    s01 = s01 + (t0 & t1)

                def byte_total(v):
                    return lax.shift_right_logical(v * byte_sum, 24).astype(jnp.float32)

                f0 = byte_total(s0)
                f1 = byte_total(s1)
                f01 = byte_total(s01)
                out0 = (ch * _CHUNK_ROWS + g * _LANES + lane) * n_out
                for j in range(n_out):
                    v = coefv[pl.ds((4 * j) * 16, 16)] + coefv[pl.ds((4 * j + 1) * 16, 16)] * f0
                    v = v + coefv[pl.ds((4 * j + 2) * 16, 16)] * f1 + coefv[pl.ds((4 * j + 3) * 16, 16)] * f01
                    plsc.store_scatter(outv, [out0 + j], v)
                return 0

            lax.fori_loop(0, groups_per_chunk, group, 0)

        pltpu.sync_copy(outv, out_hbm.at[pl.ds(base * n_out, rows_per_worker * n_out)])

    return body


def kernel(input_ids, embedding_table, dense_w):
    batch, seq_len = input_ids.shape
    n_out = dense_w.shape[1]
    assert batch % (_NUM_WORKERS * _CHUNK_ROWS) == 0
    assert seq_len % 4 == 0
    rows_per_worker = batch // _NUM_WORKERS

    m = embedding_table.astype(jnp.float32) @ dense_w.astype(jnp.float32)
    a = m[0]
    b = m[1] - m[0]
    c = m[2] - m[0]
    d = m[3] - m[1] - m[2] + m[0]
    k = seq_len * a
    coef = jnp.stack([k, b, c, d], axis=0).T.reshape(4 * n_out)
    coef = jnp.broadcast_to(coef[:, None], (4 * n_out, _LANES)).reshape(-1)

    ids = input_ids.astype(jnp.int32)

    fn = pl.kernel(
        _make_body(rows_per_worker, seq_len, n_out),
        out_type=jax.ShapeDtypeStruct((batch * n_out,), jnp.float32),
        mesh=plsc.VectorSubcoreMesh(
            core_axis_name="c",
            subcore_axis_name="s",
            num_cores=_NUM_CORES,
            num_subcores=_NUM_SUBCORES,
        ),
        scratch_types=[
            pltpu.VMEM((_CHUNK_ROWS, seq_len), jnp.int32),
            pltpu.VMEM((_CHUNK_ROWS, seq_len), jnp.int32),
            pltpu.VMEM((4 * n_out * _LANES,), jnp.float32),
            pltpu.VMEM((rows_per_worker * n_out,), jnp.float32),
            pltpu.SemaphoreType.DMA,
            pltpu.SemaphoreType.DMA,
        ],
        compiler_params=pltpu.CompilerParams(
            use_tc_tiling_on_sc=False, needs_layout_passes=False
        ),
    )
    return fn(ids, coef).reshape(batch, n_out)


# speedup vs baseline: 3.2327x; 1.9460x over previous
import jax
import jax.numpy as jnp
from jax import lax
from jax.experimental import pallas as pl
from jax.experimental.pallas import tpu as pltpu
from jax.experimental.pallas import tpu_sc as plsc

_NUM_CORES = 2
_NUM_SUBCORES = 16
_NUM_WORKERS = _NUM_CORES * _NUM_SUBCORES
_LANES = 16


def _make_body(rows_per_worker, seq_len, n_out):
    groups = rows_per_worker // _LANES

    def body(idsT_hbm, coef_hbm, out_hbm, buf, coefv, outv):
        cid = lax.axis_index("c")
        sid = lax.axis_index("s")
        wid = sid * _NUM_CORES + cid
        base = wid * rows_per_worker
        pltpu.sync_copy(idsT_hbm.at[:, pl.ds(base, rows_per_worker)], buf)
        pltpu.sync_copy(coef_hbm, coefv)
        lane = lax.iota(jnp.int32, 16)

        byte_mask = jnp.full((16,), 0x01010101, jnp.int32)
        byte_sum = jnp.full((16,), 0x01010101, jnp.int32)

        def group(g, _):
            bb0 = g * _LANES
            zero = jnp.zeros((16,), jnp.int32)
            s0 = s1 = s01 = zero
            for st in range(seq_len // 4):
                l = 4 * st
                x0 = buf[l, pl.ds(bb0, 16)]
                x1 = buf[l + 1, pl.ds(bb0, 16)]
                x2 = buf[l + 2, pl.ds(bb0, 16)]
                x3 = buf[l + 3, pl.ds(bb0, 16)]
                c = x0 | (x1 << 8) | (x2 << 16) | (x3 << 24)
                t0 = c & byte_mask
                t1 = (c >> 1) & byte_mask
                s0 = s0 + t0
                s1 = s1 + t1
                s01 = s01 + (t0 & t1)

            def byte_total(v):
                return lax.shift_right_logical(v * byte_sum, 24).astype(jnp.float32)

            f0 = byte_total(s0)
            f1 = byte_total(s1)
            f01 = byte_total(s01)
            out0 = (g * _LANES + lane) * n_out
            for j in range(n_out):
                v = coefv[pl.ds((4 * j) * 16, 16)] + coefv[pl.ds((4 * j + 1) * 16, 16)] * f0
                v = v + coefv[pl.ds((4 * j + 2) * 16, 16)] * f1 + coefv[pl.ds((4 * j + 3) * 16, 16)] * f01
                plsc.store_scatter(outv, [out0 + j], v)
            return 0

        lax.fori_loop(0, groups, group, 0)
        pltpu.sync_copy(outv, out_hbm.at[pl.ds(base * n_out, rows_per_worker * n_out)])

    return body


def kernel(input_ids, embedding_table, dense_w):
    batch, seq_len = input_ids.shape
    n_out = dense_w.shape[1]
    assert batch % (_NUM_WORKERS * _LANES) == 0
    assert seq_len % 8 == 0
    rows_per_worker = batch // _NUM_WORKERS

    m = embedding_table.astype(jnp.float32) @ dense_w.astype(jnp.float32)
    a = m[0]
    b = m[1] - m[0]
    c = m[2] - m[0]
    d = m[3] - m[1] - m[2] + m[0]
    k = seq_len * a
    coef = jnp.stack([k, b, c, d], axis=0).T.reshape(4 * n_out)
    coef = jnp.broadcast_to(coef[:, None], (4 * n_out, _LANES)).reshape(-1)

    ids_t = input_ids.astype(jnp.int32).T

    fn = pl.kernel(
        _make_body(rows_per_worker, seq_len, n_out),
        out_type=jax.ShapeDtypeStruct((batch * n_out,), jnp.float32),
        mesh=plsc.VectorSubcoreMesh(
            core_axis_name="c",
            subcore_axis_name="s",
            num_cores=_NUM_CORES,
            num_subcores=_NUM_SUBCORES,
        ),
        scratch_types=[
            pltpu.VMEM((seq_len, rows_per_worker), jnp.int32),
            pltpu.VMEM((4 * n_out * _LANES,), jnp.float32),
            pltpu.VMEM((rows_per_worker * n_out,), jnp.float32),
        ],
        compiler_params=pltpu.CompilerParams(
            use_tc_tiling_on_sc=True, needs_layout_passes=False
        ),
    )
    return fn(ids_t, coef).reshape(batch, n_out)


# speedup vs baseline: 4.9580x; 1.5337x over previous
import jax
import jax.numpy as jnp
from jax import lax
from jax.experimental import pallas as pl
from jax.experimental.pallas import tpu as pltpu
from jax.experimental.pallas import tpu_sc as plsc

_NUM_CORES = 2
_NUM_SUBCORES = 16
_NUM_WORKERS = _NUM_CORES * _NUM_SUBCORES
_LANES = 16


def _make_body(rows_per_worker, seq_len, n_out):
    groups = rows_per_worker // _LANES

    def body(idsT_hbm, coef_hbm, out_hbm, buf, coefv, outv):
        cid = lax.axis_index("c")
        sid = lax.axis_index("s")
        wid = sid * _NUM_CORES + cid
        base = wid * rows_per_worker
        pltpu.sync_copy(idsT_hbm.at[:, pl.ds(base, rows_per_worker)], buf)
        pltpu.sync_copy(coef_hbm, coefv)
        lane = lax.iota(jnp.int32, 16)

        byte_mask = jnp.full((16,), 0x01010101, jnp.int32)
        byte_sum = jnp.full((16,), 0x01010101, jnp.int32)

        def group(g, _):
            bb0 = g * _LANES
            zero = jnp.zeros((16,), jnp.int32)
            s0 = s1 = s01 = zero
            for st in range(seq_len // 4):
                l = 4 * st
                x0 = buf[l, pl.ds(bb0, 16)]
                x1 = buf[l + 1, pl.ds(bb0, 16)]
                x2 = buf[l + 2, pl.ds(bb0, 16)]
                x3 = buf[l + 3, pl.ds(bb0, 16)]
                c = x0 | (x1 << 8) | (x2 << 16) | (x3 << 24)
                t0 = c & byte_mask
                t1 = (c >> 1) & byte_mask
                s0 = s0 + t0
                s1 = s1 + t1
                s01 = s01 + (t0 & t1)

            def byte_total(v):
                return lax.shift_right_logical(v * byte_sum, 24).astype(jnp.float32)

            f0 = byte_total(s0)
            f1 = byte_total(s1)
            f01 = byte_total(s01)
            bb = g * _LANES + lane
            for j in range(n_out):
                v = coefv[pl.ds((4 * j) * 16, 16)] + coefv[pl.ds((4 * j + 1) * 16, 16)] * f0
                v = v + coefv[pl.ds((4 * j + 2) * 16, 16)] * f1 + coefv[pl.ds((4 * j + 3) * 16, 16)] * f01
                plsc.store_scatter(outv, [jnp.full((16,), j, jnp.int32), bb], v)
            return 0

        lax.fori_loop(0, groups, group, 0)
        pltpu.sync_copy(outv, out_hbm.at[:, pl.ds(base, rows_per_worker)])

    return body


def kernel(input_ids, embedding_table, dense_w):
    batch, seq_len = input_ids.shape
    n_out = dense_w.shape[1]
    assert batch % (_NUM_WORKERS * _LANES) == 0
    assert seq_len % 8 == 0
    rows_per_worker = batch // _NUM_WORKERS

    m = embedding_table.astype(jnp.float32) @ dense_w.astype(jnp.float32)
    a = m[0]
    b = m[1] - m[0]
    c = m[2] - m[0]
    d = m[3] - m[1] - m[2] + m[0]
    k = seq_len * a
    coef = jnp.stack([k, b, c, d], axis=0).T.reshape(4 * n_out)
    coef = jnp.broadcast_to(coef[:, None], (4 * n_out, _LANES)).reshape(-1)

    ids_t = input_ids.astype(jnp.int32).T

    fn = pl.kernel(
        _make_body(rows_per_worker, seq_len, n_out),
        out_type=jax.ShapeDtypeStruct((n_out, batch), jnp.float32),
        mesh=plsc.VectorSubcoreMesh(
            core_axis_name="c",
            subcore_axis_name="s",
            num_cores=_NUM_CORES,
            num_subcores=_NUM_SUBCORES,
        ),
        scratch_types=[
            pltpu.VMEM((seq_len, rows_per_worker), jnp.int32),
            pltpu.VMEM((4 * n_out * _LANES,), jnp.float32),
            pltpu.VMEM((n_out, rows_per_worker), jnp.float32),
        ],
        compiler_params=pltpu.CompilerParams(
            use_tc_tiling_on_sc=True, needs_layout_passes=False
        ),
    )
    return fn(ids_t, coef).T
